# Initial kernel scaffold; baseline (speedup 1.0000x reference)
import jax
import jax.numpy as jnp
from jax.experimental import pallas as pl


def kernel(new_case, time, infectious_object, occupation, infection_route, sex, phys_pos, age_grp, batch, io_table, occ_table, route_table, sex_table, age_table, W_pos, b_pos, W_time, b_time, W_case, b_case, gn_weight, gn_bias, gn_mean_scale):
    raise NotImplementedError("write your pallas kernel here")


# speedup vs baseline: 1.9680x; 1.9680x over previous
import functools

import jax
import jax.numpy as jnp
from jax.experimental import pallas as pl
from jax.experimental.pallas import tpu as pltpu

N = 100000
D = 64
NUM_GRAPHS = 512
EPS = 1e-5

B = 512
NB = (N + B - 1) // B
N_PAD = NB * B

V_IO = 1024
V_OCC = 512
V_ROUTE = 64
V_AGE = 16


def _onehot(idx_col, v):
    iota = jax.lax.broadcasted_iota(jnp.int32, (B, v), 1)
    return (iota == idx_col).astype(jnp.float32)


def _pass_a_kernel(io_ref, occ_ref, route_ref, age_ref, sex_ref, batch_ref,
                   nc_ref, t_ref, px_ref, py_ref,
                   io_t, occ_t, route_t, age_t, sex_t,
                   wrows, brows, gnrows,
                   x_out, a_out, c_out,
                   s_sum, s_sq, s_cnt):
    i = pl.program_id(0)

    @pl.when(i == 0)
    def _init():
        s_sum[...] = jnp.zeros_like(s_sum)
        s_sq[...] = jnp.zeros_like(s_sq)
        s_cnt[...] = jnp.zeros_like(s_cnt)

    io_col = io_ref[...].reshape(B, 1)
    occ_col = occ_ref[...].reshape(B, 1)
    route_col = route_ref[...].reshape(B, 1)
    age_col = age_ref[...].reshape(B, 1)
    sex_col = sex_ref[...].reshape(B, 1).astype(jnp.float32)

    dot = functools.partial(jax.lax.dot_general,
                            dimension_numbers=(((1,), (0,)), ((), ())),
                            preferred_element_type=jnp.float32)

    x = dot(_onehot(io_col, V_IO), io_t[...])
    x += dot(_onehot(occ_col, V_OCC), occ_t[...])
    x += dot(_onehot(route_col, V_ROUTE), route_t[...])
    x += dot(_onehot(age_col, V_AGE), age_t[...])

    s0 = sex_t[0:1, :]
    x += s0 + sex_col * (sex_t[1:2, :] - s0)

    nc = nc_ref[...].reshape(B, 1)
    t = t_ref[...].reshape(B, 1)
    px = px_ref[...].reshape(B, 1)
    py = py_ref[...].reshape(B, 1)
    x += nc * wrows[0:1, :] + t * wrows[1:2, :]
    x += px * wrows[2:3, :] + py * wrows[3:4, :]
    x += brows[0:1, :] + brows[1:2, :] + brows[2:3, :]

    x_out[...] = x

    batch_row = batch_ref[...].reshape(1, B)
    giota = jax.lax.broadcasted_iota(jnp.int32, (NUM_GRAPHS, B), 0)
    ohT = (giota == batch_row).astype(jnp.float32)
    s_sum[...] += dot(ohT, x)
    s_sq[...] += dot(ohT, x * x)
    s_cnt[...] += jnp.sum(ohT, axis=1, keepdims=True)

    @pl.when(i == NB - 1)
    def _finalize():
        cnt = jnp.maximum(s_cnt[...], 1.0)
        inv = 1.0 / cnt
        mean = s_sum[...] * inv
        ms = gnrows[0:1, :]
        var = s_sq[...] * inv - mean * mean * ms * (2.0 - ms)
        rstd = jax.lax.rsqrt(var + EPS)
        a = gnrows[1:2, :] * rstd
        a_out[...] = a
        c_out[...] = gnrows[2:3, :] - mean * ms * a


def _pass_b_kernel(x_ref, batch_ref, a_ref, c_ref, out_ref):
    batch_col = batch_ref[...].reshape(B, 1)
    oh = _onehot(batch_col, NUM_GRAPHS)
    dot = functools.partial(jax.lax.dot_general,
                            dimension_numbers=(((1,), (0,)), ((), ())),
                            preferred_element_type=jnp.float32)
    out_ref[...] = dot(oh, a_ref[...]) * x_ref[...] + dot(oh, c_ref[...])


def _pad_col(v, fill=0):
    v = jnp.pad(v, (0, N_PAD - N), constant_values=fill)
    return v.reshape(NB, B, 1)


def _pad_rows(m, rows):
    return jnp.pad(m, ((0, rows - m.shape[0]), (0, 0)))


def kernel(new_case, time, infectious_object, occupation, infection_route,
           sex, phys_pos, age_grp, batch,
           io_table, occ_table, route_table, sex_table, age_table,
           W_pos, b_pos, W_time, b_time, W_case, b_case,
           gn_weight, gn_bias, gn_mean_scale):
    f32 = jnp.float32
    i32 = jnp.int32

    io_col = _pad_col(infectious_object.astype(i32))
    occ_col = _pad_col(occupation.astype(i32))
    route_col = _pad_col(infection_route.astype(i32))
    age_col = _pad_col(age_grp.astype(i32))
    sex_col = _pad_col(sex.astype(i32))
    batch_i32 = batch.astype(i32)
    batch_row = jnp.pad(batch_i32, (0, N_PAD - N),
                        constant_values=NUM_GRAPHS).reshape(NB, 1, B)
    batch_col = jnp.pad(batch_i32, (0, N_PAD - N),
                        constant_values=NUM_GRAPHS).reshape(NB, B, 1)

    nc_col = _pad_col(new_case.astype(f32))
    t_col = _pad_col(time.astype(f32))
    px_col = _pad_col(phys_pos[:, 0].astype(f32))
    py_col = _pad_col(phys_pos[:, 1].astype(f32))

    io_t = _pad_rows(io_table.astype(f32), V_IO)
    occ_t = _pad_rows(occ_table.astype(f32), V_OCC)
    route_t = _pad_rows(route_table.astype(f32), V_ROUTE)
    age_t = age_table.astype(f32)
    sex_t = _pad_rows(sex_table.astype(f32), 8)

    wrows = _pad_rows(jnp.stack([W_case[0], W_time[0], W_pos[0], W_pos[1]]
                                ).astype(f32), 8)
    brows = _pad_rows(jnp.stack([b_case, b_time, b_pos]).astype(f32), 8)
    gnrows = _pad_rows(jnp.stack([gn_mean_scale, gn_weight, gn_bias]
                                 ).astype(f32), 8)

    col_spec = pl.BlockSpec((1, B, 1), lambda i: (i, 0, 0))
    row_spec = pl.BlockSpec((1, 1, B), lambda i: (i, 0, 0))

    def full(shape):
        return pl.BlockSpec(shape, lambda i: tuple(0 for _ in shape))

    x, a_mat, c_mat = pl.pallas_call(
        _pass_a_kernel,
        grid=(NB,),
        in_specs=[col_spec, col_spec, col_spec, col_spec, col_spec,
                  row_spec,
                  col_spec, col_spec, col_spec, col_spec,
                  full((V_IO, D)), full((V_OCC, D)), full((V_ROUTE, D)),
                  full((V_AGE, D)), full((8, D)),
                  full((8, D)), full((8, D)), full((8, D))],
        out_specs=[pl.BlockSpec((B, D), lambda i: (i, 0)),
                   full((NUM_GRAPHS, D)), full((NUM_GRAPHS, D))],
        out_shape=[jax.ShapeDtypeStruct((N_PAD, D), f32),
                   jax.ShapeDtypeStruct((NUM_GRAPHS, D), f32),
                   jax.ShapeDtypeStruct((NUM_GRAPHS, D), f32)],
        scratch_shapes=[pltpu.VMEM((NUM_GRAPHS, D), f32),
                        pltpu.VMEM((NUM_GRAPHS, D), f32),
                        pltpu.VMEM((NUM_GRAPHS, 1), f32)],
    )(io_col, occ_col, route_col, age_col, sex_col, batch_row,
      nc_col, t_col, px_col, py_col,
      io_t, occ_t, route_t, age_t, sex_t,
      wrows, brows, gnrows)

    out = pl.pallas_call(
        _pass_b_kernel,
        grid=(NB,),
        in_specs=[pl.BlockSpec((B, D), lambda i: (i, 0)),
                  col_spec,
                  full((NUM_GRAPHS, D)), full((NUM_GRAPHS, D))],
        out_specs=pl.BlockSpec((B, D), lambda i: (i, 0)),
        out_shape=jax.ShapeDtypeStruct((N_PAD, D), f32),
    )(x, batch_col, a_mat, c_mat)

    return out[:N]


# speedup vs baseline: 4.9811x; 2.5310x over previous
import functools

import jax
import jax.numpy as jnp
from jax.experimental import pallas as pl
from jax.experimental.pallas import tpu as pltpu

N = 100000
D = 64
NUM_GRAPHS = 512
EPS = 1e-5

B = 512
NB = (N + B - 1) // B
N_PAD = NB * B

V_IO = 1024
V_OCC = 512
V_ROUTE = 64
V_AGE = 16

BF = jnp.bfloat16

_dot = functools.partial(jax.lax.dot_general,
                         dimension_numbers=(((1,), (0,)), ((), ())),
                         preferred_element_type=jnp.float32)
_dot_tb = functools.partial(jax.lax.dot_general,
                            dimension_numbers=(((1,), (1,)), ((), ())),
                            preferred_element_type=jnp.float32)


def _onehot_t(idx_row, v):
    iota = jax.lax.broadcasted_iota(jnp.int32, (v, B), 0)
    return (iota == idx_row).astype(BF)


def _pass_a_kernel(idx_ref, f_ref,
                   io_t, occ_t, route_t, age_t, w_t, gcols,
                   x_out, a_out, c_out,
                   s_sum, s_sq, s_cnt):
    i = pl.program_id(0)

    @pl.when(i == 0)
    def _init():
        s_sum[...] = jnp.zeros_like(s_sum)
        s_sq[...] = jnp.zeros_like(s_sq)
        s_cnt[...] = jnp.zeros_like(s_cnt)

    idx = idx_ref[0]
    x = _dot(io_t[...], _onehot_t(idx[0:1], V_IO))
    x += _dot(occ_t[...], _onehot_t(idx[1:2], V_OCC))
    x += _dot(route_t[...], _onehot_t(idx[2:3], V_ROUTE))
    x += _dot(age_t[...], _onehot_t(idx[3:4], V_AGE))
    x += _dot(w_t[...], f_ref[0])

    x_out[...] = x.astype(BF)

    ohg = _onehot_t(idx[4:5], NUM_GRAPHS)
    s_sum[...] += _dot_tb(x.astype(BF), ohg)
    s_sq[...] += _dot_tb((x * x).astype(BF), ohg)
    s_cnt[0:1, :] += _dot_tb(jnp.ones((1, B), BF), ohg)

    @pl.when(i == NB - 1)
    def _finalize():
        cnt = jnp.maximum(s_cnt[0:1, :], 1.0)
        inv = 1.0 / cnt
        mean = s_sum[...] * inv
        ms = gcols[:, 0:1]
        var = s_sq[...] * inv - mean * mean * ms * (2.0 - ms)
        rstd = jax.lax.rsqrt(var + EPS)
        a = gcols[:, 1:2] * rstd
        a_out[...] = a.astype(BF)
        c_out[...] = (gcols[:, 2:3] - mean * ms * a).astype(BF)


def _pass_b_kernel(x_ref, idx_ref, a_ref, c_ref, out_ref):
    ohg = _onehot_t(idx_ref[0, 4:5], NUM_GRAPHS)
    ag = _dot(a_ref[...], ohg)
    cg = _dot(c_ref[...], ohg)
    out_t = ag * x_ref[...].astype(jnp.float32) + cg
    out_ref[...] = jnp.transpose(out_t, (1, 0))


def kernel(new_case, time, infectious_object, occupation, infection_route,
           sex, phys_pos, age_grp, batch,
           io_table, occ_table, route_table, sex_table, age_table,
           W_pos, b_pos, W_time, b_time, W_case, b_case,
           gn_weight, gn_bias, gn_mean_scale):
    f32 = jnp.float32
    i32 = jnp.int32

    def pad_n(a, fill=0):
        return jnp.pad(a, (0, N_PAD - N), constant_values=fill)

    zi = jnp.zeros((N_PAD,), i32)
    idx_rows = jnp.stack([
        pad_n(infectious_object.astype(i32)),
        pad_n(occupation.astype(i32)),
        pad_n(infection_route.astype(i32)),
        pad_n(age_grp.astype(i32)),
        pad_n(batch.astype(i32), NUM_GRAPHS),
        zi, zi, zi,
    ]).reshape(8, NB, B).swapaxes(0, 1)

    zf = jnp.zeros((N_PAD,), BF)
    fvals = jnp.stack([
        pad_n(new_case.astype(BF)),
        pad_n(time.astype(BF)),
        pad_n(phys_pos[:, 0].astype(BF)),
        pad_n(phys_pos[:, 1].astype(BF)),
        pad_n(sex.astype(BF)),
        jnp.ones((N_PAD,), BF),
        zf, zf,
    ]).reshape(8, NB, B).swapaxes(0, 1)

    def tpadT(tbl, v):
        return jnp.pad(tbl, ((0, v - tbl.shape[0]), (0, 0))).astype(BF).T

    io_t = tpadT(io_table.astype(f32), V_IO)
    occ_t = tpadT(occ_table.astype(f32), V_OCC)
    route_t = tpadT(route_table.astype(f32), V_ROUTE)
    age_t = tpadT(age_table.astype(f32), V_AGE)

    w_t = jnp.stack([
        W_case[0], W_time[0], W_pos[0], W_pos[1],
        sex_table[1] - sex_table[0],
        b_case + b_time + b_pos + sex_table[0],
        jnp.zeros((D,), f32), jnp.zeros((D,), f32),
    ]).astype(BF).T

    gcols = jnp.pad(jnp.stack([gn_mean_scale, gn_weight, gn_bias]),
                    ((0, 5), (0, 0))).astype(f32).T

    idx_spec = pl.BlockSpec((1, 8, B), lambda i: (i, 0, 0))

    def full(shape):
        return pl.BlockSpec(shape, lambda i: tuple(0 for _ in shape))

    x_t, a_mat, c_mat = pl.pallas_call(
        _pass_a_kernel,
        grid=(NB,),
        in_specs=[idx_spec, idx_spec,
                  full((D, V_IO)), full((D, V_OCC)), full((D, V_ROUTE)),
                  full((D, V_AGE)), full((D, 8)), full((D, 8))],
        out_specs=[pl.BlockSpec((D, B), lambda i: (0, i)),
                   full((D, NUM_GRAPHS)), full((D, NUM_GRAPHS))],
        out_shape=[jax.ShapeDtypeStruct((D, N_PAD), BF),
                   jax.ShapeDtypeStruct((D, NUM_GRAPHS), BF),
                   jax.ShapeDtypeStruct((D, NUM_GRAPHS), BF)],
        scratch_shapes=[pltpu.VMEM((D, NUM_GRAPHS), f32),
                        pltpu.VMEM((D, NUM_GRAPHS), f32),
                        pltpu.VMEM((8, NUM_GRAPHS), f32)],
    )(idx_rows, fvals, io_t, occ_t, route_t, age_t, w_t, gcols)

    out = pl.pallas_call(
        _pass_b_kernel,
        grid=(NB,),
        in_specs=[pl.BlockSpec((D, B), lambda i: (0, i)),
                  idx_spec,
                  full((D, NUM_GRAPHS)), full((D, NUM_GRAPHS))],
        out_specs=pl.BlockSpec((B, D), lambda i: (i, 0)),
        out_shape=jax.ShapeDtypeStruct((N_PAD, D), f32),
    )(x_t, idx_rows, a_mat, c_mat)

    return out[:N]


# speedup vs baseline: 5.3593x; 1.0759x over previous
import functools

import jax
import jax.numpy as jnp
from jax.experimental import pallas as pl
from jax.experimental.pallas import tpu as pltpu

N = 100000
D = 64
NUM_GRAPHS = 512
EPS = 1e-5

B = 512
NB = (N + B - 1) // B
N_PAD = NB * B

V_IO = 1024
V_OCC = 512
V_ROUTE = 64
V_AGE = 16

BF = jnp.bfloat16

_dot = functools.partial(jax.lax.dot_general,
                         dimension_numbers=(((1,), (0,)), ((), ())),
                         preferred_element_type=jnp.float32)
_dot_tb = functools.partial(jax.lax.dot_general,
                            dimension_numbers=(((1,), (1,)), ((), ())),
                            preferred_element_type=jnp.float32)


def _onehot_t(idx_row, v):
    iota = jax.lax.broadcasted_iota(jnp.int32, (v, B), 0)
    return (iota == idx_row).astype(BF)


def _pass_a_kernel(idx_ref, f_ref,
                   io_t, occ_t, route_t, age_t, w_t, gcols,
                   x_out, a_out, c_out,
                   s_sum, s_sq, s_cnt):
    i = pl.program_id(0)

    @pl.when(i == 0)
    def _init():
        s_sum[...] = jnp.zeros_like(s_sum)
        s_sq[...] = jnp.zeros_like(s_sq)
        s_cnt[...] = jnp.zeros_like(s_cnt)

    idx = idx_ref[...]
    x = _dot(io_t[...], _onehot_t(idx[0:1], V_IO))
    x += _dot(occ_t[...], _onehot_t(idx[1:2], V_OCC))
    x += _dot(route_t[...], _onehot_t(idx[2:3], V_ROUTE))
    x += _dot(age_t[...], _onehot_t(idx[3:4], V_AGE))
    x += _dot(w_t[...], f_ref[...])

    x_out[...] = x.astype(BF)

    ohg = _onehot_t(idx[4:5], NUM_GRAPHS)
    s_sum[...] += _dot_tb(x.astype(BF), ohg)
    s_sq[...] += _dot_tb((x * x).astype(BF), ohg)
    s_cnt[0:1, :] += _dot_tb(jnp.ones((1, B), BF), ohg)

    @pl.when(i == NB - 1)
    def _finalize():
        cnt = jnp.maximum(s_cnt[0:1, :], 1.0)
        inv = 1.0 / cnt
        mean = s_sum[...] * inv
        ms = gcols[:, 0:1]
        var = s_sq[...] * inv - mean * mean * ms * (2.0 - ms)
        rstd = jax.lax.rsqrt(var + EPS)
        a = gcols[:, 1:2] * rstd
        a_out[...] = a.astype(BF)
        c_out[...] = (gcols[:, 2:3] - mean * ms * a).astype(BF)


def _pass_b_kernel(x_ref, idx_ref, a_ref, c_ref, out_ref):
    ohg = _onehot_t(idx_ref[4:5], NUM_GRAPHS)
    ag = _dot(a_ref[...], ohg)
    cg = _dot(c_ref[...], ohg)
    out_t = ag * x_ref[...].astype(jnp.float32) + cg
    out_ref[...] = jnp.transpose(out_t, (1, 0))


def kernel(new_case, time, infectious_object, occupation, infection_route,
           sex, phys_pos, age_grp, batch,
           io_table, occ_table, route_table, sex_table, age_table,
           W_pos, b_pos, W_time, b_time, W_case, b_case,
           gn_weight, gn_bias, gn_mean_scale):
    f32 = jnp.float32
    i32 = jnp.int32

    def pad_n(a, fill=0):
        return jnp.pad(a, (0, N_PAD - N), constant_values=fill)

    zi = jnp.zeros((N_PAD,), i32)
    idx_rows = jnp.stack([
        pad_n(infectious_object.astype(i32)),
        pad_n(occupation.astype(i32)),
        pad_n(infection_route.astype(i32)),
        pad_n(age_grp.astype(i32)),
        pad_n(batch.astype(i32), NUM_GRAPHS),
        zi, zi, zi,
    ])

    zf = jnp.zeros((N_PAD,), BF)
    fvals = jnp.stack([
        pad_n(new_case.astype(BF)),
        pad_n(time.astype(BF)),
        pad_n(phys_pos[:, 0].astype(BF)),
        pad_n(phys_pos[:, 1].astype(BF)),
        pad_n(sex.astype(BF)),
        jnp.ones((N_PAD,), BF),
        zf, zf,
    ])

    def tpadT(tbl, v):
        return jnp.pad(tbl, ((0, v - tbl.shape[0]), (0, 0))).astype(BF).T

    io_t = tpadT(io_table.astype(f32), V_IO)
    occ_t = tpadT(occ_table.astype(f32), V_OCC)
    route_t = tpadT(route_table.astype(f32), V_ROUTE)
    age_t = tpadT(age_table.astype(f32), V_AGE)

    w_t = jnp.stack([
        W_case[0], W_time[0], W_pos[0], W_pos[1],
        sex_table[1] - sex_table[0],
        b_case + b_time + b_pos + sex_table[0],
        jnp.zeros((D,), f32), jnp.zeros((D,), f32),
    ]).astype(BF).T

    gcols = jnp.pad(jnp.stack([gn_mean_scale, gn_weight, gn_bias]),
                    ((0, 5), (0, 0))).astype(f32).T

    idx_spec = pl.BlockSpec((8, B), lambda i: (0, i))

    def full(shape):
        return pl.BlockSpec(shape, lambda i: tuple(0 for _ in shape))

    x_t, a_mat, c_mat = pl.pallas_call(
        _pass_a_kernel,
        grid=(NB,),
        in_specs=[idx_spec, idx_spec,
                  full((D, V_IO)), full((D, V_OCC)), full((D, V_ROUTE)),
                  full((D, V_AGE)), full((D, 8)), full((D, 8))],
        out_specs=[pl.BlockSpec((D, B), lambda i: (0, i)),
                   full((D, NUM_GRAPHS)), full((D, NUM_GRAPHS))],
        out_shape=[jax.ShapeDtypeStruct((D, N_PAD), BF),
                   jax.ShapeDtypeStruct((D, NUM_GRAPHS), BF),
                   jax.ShapeDtypeStruct((D, NUM_GRAPHS), BF)],
        scratch_shapes=[pltpu.VMEM((D, NUM_GRAPHS), f32),
                        pltpu.VMEM((D, NUM_GRAPHS), f32),
                        pltpu.VMEM((8, NUM_GRAPHS), f32)],
    )(idx_rows, fvals, io_t, occ_t, route_t, age_t, w_t, gcols)

    out = pl.pallas_call(
        _pass_b_kernel,
        grid=(NB,),
        in_specs=[pl.BlockSpec((D, B), lambda i: (0, i)),
                  idx_spec,
                  full((D, NUM_GRAPHS)), full((D, NUM_GRAPHS))],
        out_specs=pl.BlockSpec((B, D), lambda i: (i, 0)),
        out_shape=jax.ShapeDtypeStruct((N, D), f32),
    )(x_t, idx_rows, a_mat, c_mat)

    return out


# speedup vs baseline: 6.1631x; 1.1500x over previous
import functools

import jax
import jax.numpy as jnp
from jax.experimental import pallas as pl
from jax.experimental.pallas import tpu as pltpu

N = 100000
D = 64
NUM_GRAPHS = 512
EPS = 1e-5

B = 1024
NB = 98
N_PAD = NB * B

V_IO = 1024
V_OCC = 512
V_ROUTE = 64
V_AGE = 16

BF = jnp.bfloat16
I16 = jnp.int16

_dot = functools.partial(jax.lax.dot_general,
                         dimension_numbers=(((1,), (0,)), ((), ())),
                         preferred_element_type=jnp.float32)
_dot_tb = functools.partial(jax.lax.dot_general,
                            dimension_numbers=(((1,), (1,)), ((), ())),
                            preferred_element_type=jnp.float32)


def _onehot_t(idx_row, v):
    iota = jax.lax.broadcasted_iota(I16, (v, B), 0)
    return (iota == idx_row).astype(BF)


def _pass_a_kernel(idx_ref, f_ref,
                   io_t, occ_t, route_t, age_t, w_t, gcols,
                   x_out, ac_out,
                   s_all):
    i = pl.program_id(0)

    @pl.when(i == 0)
    def _init():
        s_all[...] = jnp.zeros_like(s_all)

    idx = idx_ref[...]
    x = _dot(io_t[...], _onehot_t(idx[0:1], V_IO))
    x += _dot(occ_t[...], _onehot_t(idx[1:2], V_OCC))
    x += _dot(route_t[...], _onehot_t(idx[2:3], V_ROUTE))
    x += _dot(age_t[...], _onehot_t(idx[3:4], V_AGE))
    x += _dot(w_t[...], f_ref[...])

    x_out[...] = x.astype(BF)

    ohg = _onehot_t(idx[4:5], NUM_GRAPHS)
    xs = jnp.concatenate([x.astype(BF), (x * x).astype(BF),
                          jnp.ones((8, B), BF)], axis=0)
    s_all[...] += _dot_tb(xs, ohg)

    @pl.when(i == NB - 1)
    def _finalize():
        cnt = jnp.maximum(s_all[2 * D:2 * D + 1, :], 1.0)
        inv = 1.0 / cnt
        mean = s_all[0:D, :] * inv
        ms = gcols[:, 0:1]
        var = s_all[D:2 * D, :] * inv - mean * mean * ms * (2.0 - ms)
        rstd = jax.lax.rsqrt(var + EPS)
        a = gcols[:, 1:2] * rstd
        ac_out[0:D, :] = a.astype(BF)
        ac_out[D:2 * D, :] = (gcols[:, 2:3] - mean * ms * a).astype(BF)


def _pass_b_kernel(x_ref, idx_ref, ac_ref, out_ref):
    ohg = _onehot_t(idx_ref[4:5], NUM_GRAPHS)
    acg = _dot(ac_ref[...], ohg)
    out_t = acg[0:D, :] * x_ref[...].astype(jnp.float32) + acg[D:2 * D, :]
    out_ref[...] = jnp.transpose(out_t, (1, 0))


def kernel(new_case, time, infectious_object, occupation, infection_route,
           sex, phys_pos, age_grp, batch,
           io_table, occ_table, route_table, sex_table, age_table,
           W_pos, b_pos, W_time, b_time, W_case, b_case,
           gn_weight, gn_bias, gn_mean_scale):
    f32 = jnp.float32

    def pad_n(a, fill=0):
        return jnp.pad(a, (0, N_PAD - N), constant_values=fill)

    zi = jnp.zeros((N_PAD,), I16)
    idx_rows = jnp.stack([
        pad_n(infectious_object.astype(I16)),
        pad_n(occupation.astype(I16)),
        pad_n(infection_route.astype(I16)),
        pad_n(age_grp.astype(I16)),
        pad_n(batch.astype(I16), NUM_GRAPHS),
        zi, zi, zi,
    ])

    zf = jnp.zeros((N_PAD,), BF)
    fvals = jnp.stack([
        pad_n(new_case.astype(BF)),
        pad_n(time.astype(BF)),
        pad_n(phys_pos[:, 0].astype(BF)),
        pad_n(phys_pos[:, 1].astype(BF)),
        pad_n(sex.astype(BF)),
        jnp.ones((N_PAD,), BF),
        zf, zf,
    ])

    def tpadT(tbl, v):
        return jnp.pad(tbl, ((0, v - tbl.shape[0]), (0, 0))).astype(BF).T

    io_t = tpadT(io_table.astype(f32), V_IO)
    occ_t = tpadT(occ_table.astype(f32), V_OCC)
    route_t = tpadT(route_table.astype(f32), V_ROUTE)
    age_t = tpadT(age_table.astype(f32), V_AGE)

    w_t = jnp.stack([
        W_case[0], W_time[0], W_pos[0], W_pos[1],
        sex_table[1] - sex_table[0],
        b_case + b_time + b_pos + sex_table[0],
        jnp.zeros((D,), f32), jnp.zeros((D,), f32),
    ]).astype(BF).T

    gcols = jnp.pad(jnp.stack([gn_mean_scale, gn_weight, gn_bias]),
                    ((0, 5), (0, 0))).astype(f32).T

    idx_spec = pl.BlockSpec((8, B), lambda i: (0, i))

    def full(shape):
        return pl.BlockSpec(shape, lambda i: tuple(0 for _ in shape))

    x_t, ac_mat = pl.pallas_call(
        _pass_a_kernel,
        grid=(NB,),
        in_specs=[idx_spec, idx_spec,
                  full((D, V_IO)), full((D, V_OCC)), full((D, V_ROUTE)),
                  full((D, V_AGE)), full((D, 8)), full((D, 8))],
        out_specs=[pl.BlockSpec((D, B), lambda i: (0, i)),
                   full((2 * D, NUM_GRAPHS))],
        out_shape=[jax.ShapeDtypeStruct((D, N_PAD), BF),
                   jax.ShapeDtypeStruct((2 * D, NUM_GRAPHS), BF)],
        scratch_shapes=[pltpu.VMEM((2 * D + 8, NUM_GRAPHS), f32)],
    )(idx_rows, fvals, io_t, occ_t, route_t, age_t, w_t, gcols)

    out = pl.pallas_call(
        _pass_b_kernel,
        grid=(NB,),
        in_specs=[pl.BlockSpec((D, B), lambda i: (0, i)),
                  idx_spec,
                  full((2 * D, NUM_GRAPHS))],
        out_specs=pl.BlockSpec((B, D), lambda i: (i, 0)),
        out_shape=jax.ShapeDtypeStruct((N, D), f32),
    )(x_t, idx_rows, ac_mat)

    return out


# speedup vs baseline: 7.8584x; 1.2751x over previous
import functools

import jax
import jax.numpy as jnp
from jax.experimental import pallas as pl
from jax.experimental.pallas import tpu as pltpu

N = 100000
D = 64
NUM_GRAPHS = 512
EPS = 1e-5

B = 1024
NB = 98
N_PAD = NB * B

V_IO = 1024
V_OCC = 512
V_ROUTE = 64
V_AGE = 16

BF = jnp.bfloat16
I16 = jnp.int16

_dot = functools.partial(jax.lax.dot_general,
                         dimension_numbers=(((1,), (0,)), ((), ())),
                         preferred_element_type=jnp.float32)
_dot_tb = functools.partial(jax.lax.dot_general,
                            dimension_numbers=(((1,), (1,)), ((), ())),
                            preferred_element_type=jnp.float32)


def _onehot_t(idx_row, v):
    iota = jax.lax.broadcasted_iota(jnp.int32, (v, B), 0)
    return (iota == idx_row).astype(BF)


def _pass_a_kernel(idx_ref, f_ref,
                   io_t, occ_t, route_t, age_t, w_t, gcols,
                   x_out, ac_out,
                   s_all):
    i = pl.program_id(0)

    @pl.when(i == 0)
    def _init():
        s_all[...] = jnp.zeros_like(s_all)

    idx = idx_ref[...]
    x = _dot(io_t[...], _onehot_t(idx[0:1], V_IO))
    x += _dot(occ_t[...], _onehot_t(idx[1:2], V_OCC))
    x += _dot(route_t[...], _onehot_t(idx[2:3], V_ROUTE))
    x += _dot(age_t[...], _onehot_t(idx[3:4], V_AGE))
    x += _dot(w_t[...], f_ref[...])

    x_out[...] = x.astype(BF)

    ohg = _onehot_t(idx[4:5], NUM_GRAPHS)
    xs = jnp.concatenate([x.astype(BF), (x * x).astype(BF),
                          jnp.ones((8, B), BF)], axis=0)
    s_all[...] += _dot_tb(xs, ohg)

    @pl.when(i == NB - 1)
    def _finalize():
        cnt = jnp.maximum(s_all[2 * D:2 * D + 1, :], 1.0)
        inv = 1.0 / cnt
        mean = s_all[0:D, :] * inv
        ms = gcols[:, 0:1]
        var = s_all[D:2 * D, :] * inv - mean * mean * ms * (2.0 - ms)
        rstd = jax.lax.rsqrt(var + EPS)
        a = gcols[:, 1:2] * rstd
        ac_out[0:D, :] = a.astype(BF)
        ac_out[D:2 * D, :] = (gcols[:, 2:3] - mean * ms * a).astype(BF)


def _pass_b_kernel(x_ref, idx_ref, ac_ref, out_ref):
    ohg = _onehot_t(idx_ref[4:5], NUM_GRAPHS)
    acg = _dot(ac_ref[...], ohg)
    out_t = acg[0:D, :] * x_ref[...].astype(jnp.float32) + acg[D:2 * D, :]
    out_ref[...] = jnp.transpose(out_t, (1, 0))


def kernel(new_case, time, infectious_object, occupation, infection_route,
           sex, phys_pos, age_grp, batch,
           io_table, occ_table, route_table, sex_table, age_table,
           W_pos, b_pos, W_time, b_time, W_case, b_case,
           gn_weight, gn_bias, gn_mean_scale):
    f32 = jnp.float32

    def pad_n(a, fill=0):
        return jnp.pad(a, (0, N_PAD - N), constant_values=fill)

    i32 = jnp.int32
    zi = jnp.zeros((N_PAD,), i32)
    idx_rows = jnp.stack([
        pad_n(infectious_object.astype(i32)),
        pad_n(occupation.astype(i32)),
        pad_n(infection_route.astype(i32)),
        pad_n(age_grp.astype(i32)),
        pad_n(batch.astype(i32), NUM_GRAPHS),
        zi, zi, zi,
    ])

    zf = jnp.zeros((N_PAD,), BF)
    fvals = jnp.stack([
        pad_n(new_case.astype(BF)),
        pad_n(time.astype(BF)),
        pad_n(phys_pos[:, 0].astype(BF)),
        pad_n(phys_pos[:, 1].astype(BF)),
        pad_n(sex.astype(BF)),
        jnp.ones((N_PAD,), BF),
        zf, zf,
    ])

    def tpadT(tbl, v):
        return jnp.pad(tbl, ((0, v - tbl.shape[0]), (0, 0))).astype(BF).T

    io_t = tpadT(io_table.astype(f32), V_IO)
    occ_t = tpadT(occ_table.astype(f32), V_OCC)
    route_t = tpadT(route_table.astype(f32), V_ROUTE)
    age_t = tpadT(age_table.astype(f32), V_AGE)

    w_t = jnp.stack([
        W_case[0], W_time[0], W_pos[0], W_pos[1],
        sex_table[1] - sex_table[0],
        b_case + b_time + b_pos + sex_table[0],
        jnp.zeros((D,), f32), jnp.zeros((D,), f32),
    ]).astype(BF).T

    gcols = jnp.pad(jnp.stack([gn_mean_scale, gn_weight, gn_bias]),
                    ((0, 5), (0, 0))).astype(f32).T

    idx_spec = pl.BlockSpec((8, B), lambda i: (0, i))

    def full(shape):
        return pl.BlockSpec(shape, lambda i: tuple(0 for _ in shape))

    x_t, ac_mat = pl.pallas_call(
        _pass_a_kernel,
        grid=(NB,),
        in_specs=[idx_spec, idx_spec,
                  full((D, V_IO)), full((D, V_OCC)), full((D, V_ROUTE)),
                  full((D, V_AGE)), full((D, 8)), full((D, 8))],
        out_specs=[pl.BlockSpec((D, B), lambda i: (0, i)),
                   full((2 * D, NUM_GRAPHS))],
        out_shape=[jax.ShapeDtypeStruct((D, N_PAD), BF),
                   jax.ShapeDtypeStruct((2 * D, NUM_GRAPHS), BF)],
        scratch_shapes=[pltpu.VMEM((2 * D + 8, NUM_GRAPHS), f32)],
    )(idx_rows, fvals, io_t, occ_t, route_t, age_t, w_t, gcols)

    out = pl.pallas_call(
        _pass_b_kernel,
        grid=(NB,),
        in_specs=[pl.BlockSpec((D, B), lambda i: (0, i)),
                  idx_spec,
                  full((2 * D, NUM_GRAPHS))],
        out_specs=pl.BlockSpec((B, D), lambda i: (i, 0)),
        out_shape=jax.ShapeDtypeStruct((N, D), f32),
    )(x_t, idx_rows, ac_mat)

    return out
